# baseline (speedup 1.0000x reference)
import functools

import jax
import jax.numpy as jnp
from jax import lax
from jax.experimental import pallas as pl
from jax.experimental.pallas import tpu as pltpu
from jax.experimental.pallas import tpu_sc as plsc

NUM_EXPERTS = 8
TOP_K = 2
LANES = 16
NUM_WORKERS = 32
TBLK = 1024


def _gate_logits_body(w_ref, x_ref, out_ref):
    out_ref[...] = lax.dot_general(
        w_ref[...], x_ref[...],
        dimension_numbers=(((1,), (1,)), ((), ())),
        preferred_element_type=jnp.float32)


def _gate_logits(x, w, tokens, chunk=0, nchunks=1):
    hidden = x.shape[1]
    ctok = tokens // nchunks
    nblk = ctok // TBLK
    return pl.pallas_call(
        _gate_logits_body,
        grid=(nblk,),
        in_specs=[
            pl.BlockSpec((NUM_EXPERTS, hidden), lambda i: (0, 0)),
            pl.BlockSpec((TBLK, hidden),
                         lambda i, c=chunk, n=nblk: (c * n + i, 0)),
        ],
        out_specs=pl.BlockSpec((NUM_EXPERTS, TBLK), lambda i: (0, i)),
        out_shape=jax.ShapeDtypeStruct((NUM_EXPERTS, ctok), jnp.float32),
        compiler_params=pltpu.CompilerParams(
            dimension_semantics=("arbitrary",)),
    )(w, x)


def _routing_body(tok_per_w, tokens, logits_hbm, w_hbm, e_hbm, lv, wv, ev):
    wid = lax.axis_index("s") * 2 + lax.axis_index("c")
    base = wid * tok_per_w
    pltpu.sync_copy(logits_hbm.at[:, pl.ds(base, tok_per_w)], lv)

    neg_inf = jnp.full((LANES,), -jnp.inf, jnp.float32)
    idx_c = [jnp.full((LANES,), e, jnp.int32) for e in range(NUM_EXPERTS)]

    def argmax_tree(vals, idxs):
        while len(vals) > 1:
            nv, ni = [], []
            for a in range(0, len(vals), 2):
                take = vals[a] >= vals[a + 1]
                nv.append(jnp.where(take, vals[a], vals[a + 1]))
                ni.append(jnp.where(take, idxs[a], idxs[a + 1]))
            vals, idxs = nv, ni
        return vals[0], idxs[0]

    def group(g):
        t0 = g * LANES
        ls = [lv[e, pl.ds(t0, LANES)] for e in range(NUM_EXPERTS)]
        m1, e1 = argmax_tree(ls, idx_c)
        ls2 = [jnp.where(e1 == idx_c[e], neg_inf, ls[e])
               for e in range(NUM_EXPERTS)]
        m2, e2 = argmax_tree(ls2, idx_c)
        t = jnp.exp(m2 - m1)
        w1 = 1.0 / (1.0 + t)
        w2 = t * w1
        wv[0, pl.ds(t0, LANES)] = w1
        wv[1, pl.ds(t0, LANES)] = w2
        ev[0, pl.ds(t0, LANES)] = e1
        ev[1, pl.ds(t0, LANES)] = e2

    def group2(g2, carry):
        group(g2 * 2)
        group(g2 * 2 + 1)
        return carry

    lax.fori_loop(0, tok_per_w // LANES // 2, group2, 0)
    for k in range(TOP_K):
        pltpu.sync_copy(wv.at[k, :], w_hbm.at[k, pl.ds(base, tok_per_w)])
        pltpu.sync_copy(ev.at[k, :], e_hbm.at[k, pl.ds(base, tok_per_w)])


def _routing(logits_t, tokens):
    tok_per_w = tokens // NUM_WORKERS
    mesh = plsc.VectorSubcoreMesh(core_axis_name="c", subcore_axis_name="s")
    fn = pl.kernel(
        functools.partial(_routing_body, tok_per_w, tokens),
        mesh=mesh,
        out_type=[
            jax.ShapeDtypeStruct((TOP_K, tokens), jnp.float32),
            jax.ShapeDtypeStruct((TOP_K, tokens), jnp.int32),
        ],
        scratch_types=[
            pltpu.VMEM((NUM_EXPERTS, tok_per_w), jnp.float32),
            pltpu.VMEM((TOP_K, tok_per_w), jnp.float32),
            pltpu.VMEM((TOP_K, tok_per_w), jnp.int32),
        ],
    )
    return fn(logits_t)


NCHUNKS = 2


def kernel(hidden_states, gate_weight):
    batch, seq, hidden = hidden_states.shape
    tokens = batch * seq
    x = hidden_states.reshape(tokens, hidden)
    ws, es = [], []
    for c in range(NCHUNKS):
        logits_c = _gate_logits(x, gate_weight, tokens, c, NCHUNKS)
        w_c, e_c = _routing(logits_c, tokens // NCHUNKS)
        ws.append(w_c)
        es.append(e_c)
    w_pl = jnp.concatenate(ws, axis=1) if NCHUNKS > 1 else ws[0]
    e_pl = jnp.concatenate(es, axis=1) if NCHUNKS > 1 else es[0]
    return (w_pl.T, e_pl.T)


# speedup vs baseline: 1.0875x; 1.0875x over previous
import functools

import jax
import jax.numpy as jnp
from jax import lax
from jax.experimental import pallas as pl
from jax.experimental.pallas import tpu as pltpu
from jax.experimental.pallas import tpu_sc as plsc

NUM_EXPERTS = 8
TOP_K = 2
LANES = 16
NUM_WORKERS = 32
TBLK = 1024


def _gate_logits_body(w_ref, x_ref, out_ref):
    out_ref[...] = lax.dot_general(
        w_ref[...], x_ref[...],
        dimension_numbers=(((1,), (1,)), ((), ())),
        preferred_element_type=jnp.float32)


def _gate_logits(x, w, tokens, chunk=0, nchunks=1):
    hidden = x.shape[1]
    ctok = tokens // nchunks
    nblk = ctok // TBLK
    return pl.pallas_call(
        _gate_logits_body,
        grid=(nblk,),
        in_specs=[
            pl.BlockSpec((NUM_EXPERTS, hidden), lambda i: (0, 0)),
            pl.BlockSpec((TBLK, hidden),
                         lambda i, c=chunk, n=nblk: (c * n + i, 0)),
        ],
        out_specs=pl.BlockSpec((NUM_EXPERTS, TBLK), lambda i: (0, i)),
        out_shape=jax.ShapeDtypeStruct((NUM_EXPERTS, ctok), jnp.float32),
        compiler_params=pltpu.CompilerParams(
            dimension_semantics=("arbitrary",)),
    )(w, x)


def _routing_body(tok_per_w, tokens, logits_hbm, w_hbm, e_hbm, lv, wv, ev):
    wid = lax.axis_index("s") * 2 + lax.axis_index("c")
    base = wid * tok_per_w
    pltpu.sync_copy(logits_hbm.at[:, pl.ds(base, tok_per_w)], lv)

    neg_inf = jnp.full((LANES,), -jnp.inf, jnp.float32)
    idx_c = [jnp.full((LANES,), e, jnp.int32) for e in range(NUM_EXPERTS)]

    def argmax_tree(vals, idxs):
        while len(vals) > 1:
            nv, ni = [], []
            for a in range(0, len(vals), 2):
                take = vals[a] >= vals[a + 1]
                nv.append(jnp.where(take, vals[a], vals[a + 1]))
                ni.append(jnp.where(take, idxs[a], idxs[a + 1]))
            vals, idxs = nv, ni
        return vals[0], idxs[0]

    def group(g):
        t0 = g * LANES
        ls = [lv[e, pl.ds(t0, LANES)] for e in range(NUM_EXPERTS)]
        m1, e1 = argmax_tree(ls, idx_c)
        ls2 = [jnp.where(e1 == idx_c[e], neg_inf, ls[e])
               for e in range(NUM_EXPERTS)]
        m2, e2 = argmax_tree(ls2, idx_c)
        t = jnp.exp(m2 - m1)
        w1 = 1.0 / (1.0 + t)
        w2 = t * w1
        wv[0, pl.ds(t0, LANES)] = w1
        wv[1, pl.ds(t0, LANES)] = w2
        ev[0, pl.ds(t0, LANES)] = e1
        ev[1, pl.ds(t0, LANES)] = e2

    def group2(g2, carry):
        group(g2 * 2)
        group(g2 * 2 + 1)
        return carry

    lax.fori_loop(0, tok_per_w // LANES // 2, group2, 0)
    for k in range(TOP_K):
        pltpu.sync_copy(wv.at[k, :], w_hbm.at[k, pl.ds(base, tok_per_w)])
        pltpu.sync_copy(ev.at[k, :], e_hbm.at[k, pl.ds(base, tok_per_w)])


def _routing(logits_t, tokens):
    tok_per_w = tokens // NUM_WORKERS
    mesh = plsc.VectorSubcoreMesh(core_axis_name="c", subcore_axis_name="s")
    fn = pl.kernel(
        functools.partial(_routing_body, tok_per_w, tokens),
        mesh=mesh,
        out_type=[
            jax.ShapeDtypeStruct((TOP_K, tokens), jnp.float32),
            jax.ShapeDtypeStruct((TOP_K, tokens), jnp.int32),
        ],
        scratch_types=[
            pltpu.VMEM((NUM_EXPERTS, tok_per_w), jnp.float32),
            pltpu.VMEM((TOP_K, tok_per_w), jnp.float32),
            pltpu.VMEM((TOP_K, tok_per_w), jnp.int32),
        ],
    )
    return fn(logits_t)


NCHUNKS = 1


def kernel(hidden_states, gate_weight):
    batch, seq, hidden = hidden_states.shape
    tokens = batch * seq
    x = hidden_states.reshape(tokens, hidden)
    ws, es = [], []
    for c in range(NCHUNKS):
        logits_c = _gate_logits(x, gate_weight, tokens, c, NCHUNKS)
        w_c, e_c = _routing(logits_c, tokens // NCHUNKS)
        ws.append(w_c)
        es.append(e_c)
    w_pl = jnp.concatenate(ws, axis=1) if NCHUNKS > 1 else ws[0]
    e_pl = jnp.concatenate(es, axis=1) if NCHUNKS > 1 else es[0]
    return (w_pl.T, e_pl.T)
